# baseline (speedup 1.0000x reference)
import jax
import jax.numpy as jnp
from jax import lax
from jax.experimental import pallas as pl
from jax.experimental.pallas import tpu as pltpu
from jax.experimental.pallas import tpu_sc as plsc

BATCH = 1024
DIM = 128
NUM_OUTPUTS = 16384
MULT = NUM_OUTPUTS // BATCH
ROW = DIM * MULT
NW = 32
ROWS_PER_W = BATCH // NW
L = 16
EG = 4
NG = ROWS_PER_W // EG
EGW = EG * ROW
RT = ROW + L


def _body(zm_hbm, zv_hbm, x_hbm, eps_hbm, out_hbm,
          zm_a, sf_a, epsg, xa, xb, zrow, idxb, cntb, ebuf, orow, hist, rtab,
          sem_e, sem_xa, sem_xb, sem_o):
    wid = lax.axis_index("s") * 2 + lax.axis_index("c")
    base = wid * ROWS_PER_W

    zeros = jnp.zeros((L,), jnp.float32)
    izeros = jnp.zeros((L,), jnp.int32)
    iones = jnp.full((L,), 1, jnp.int32)
    xrefs = (xa, xb)
    sem_x = (sem_xa, sem_xb)

    pltpu.sync_copy(zm_hbm.at[pl.ds(base * DIM, ROWS_PER_W * DIM)], zm_a)
    pltpu.sync_copy(zv_hbm.at[pl.ds(base * DIM, ROWS_PER_W * DIM)], sf_a)

    @plsc.parallel_loop(0, ROWS_PER_W * DIM, step=L, unroll=8)
    def _sf(off):
        sl = pl.ds(off, L)
        sf_a[sl] = jnp.exp(0.5 * sf_a[sl])

    @plsc.parallel_loop(0, NUM_OUTPUTS, step=L, unroll=8)
    def _clear(off):
        hist[pl.ds(off, L)] = izeros

    lanes = lax.iota(jnp.int32, L).astype(jnp.float32)

    @plsc.parallel_loop(0, RT, step=L, unroll=8)
    def _rt(off):
        c = jnp.full((L,), off, jnp.int32).astype(jnp.float32) + lanes
        rtab[pl.ds(off, L)] = 1.0 / c

    def _start_eps(g):
        pltpu.async_copy(eps_hbm.at[pl.ds((base + g * EG) * ROW, EGW)],
                         epsg.at[pl.ds(lax.rem(g, 2) * EGW, EGW)], sem_e)

    def _drain_eps():
        pltpu.make_async_copy(eps_hbm.at[pl.ds(0, EGW)],
                              epsg.at[pl.ds(0, EGW)], sem_e).wait()

    def _start_x(r, q):
        pltpu.async_copy(x_hbm.at[pl.ds(r * NUM_OUTPUTS, NUM_OUTPUTS)],
                         xrefs[q], sem_x[q])

    def _drain_x(q):
        pltpu.make_async_copy(x_hbm.at[pl.ds(0, NUM_OUTPUTS)], xrefs[q],
                              sem_x[q]).wait()

    def _drain_out():
        pltpu.make_async_copy(orow, out_hbm.at[0], sem_o).wait()

    _start_eps(0)
    _start_x(base, 0)

    def _row(r, j, q, out_pending):
        xr = xrefs[q]
        zm_base = j * DIM
        eps_base = lax.rem(j, 2 * EG) * ROW

        @plsc.parallel_loop(
            0, ROW, step=L, unroll=16,
            carry=(jnp.full((L,), jnp.inf, jnp.float32),
                   jnp.full((L,), -jnp.inf, jnp.float32)))
        def _pa(off, carry):
            mn, mx = carry
            dsl = pl.ds(zm_base + lax.rem(off, DIM), L)
            z = zm_a[dsl] + sf_a[dsl] * epsg[pl.ds(eps_base + off, L)]
            zrow[pl.ds(off, L)] = z
            return jnp.minimum(mn, z), jnp.maximum(mx, z)
        minv, maxv = _pa
        mn = jnp.min(minv)
        rng = jnp.maximum(jnp.max(maxv) - mn, 1e-12)
        scale = float(NUM_OUTPUTS) / jnp.full((L,), rng, jnp.float32)

        @plsc.parallel_loop(0, ROW, step=L, unroll=16)
        def _pb(off):
            sl = pl.ds(off, L)
            t = (zrow[sl] - mn) * scale
            ix = jnp.minimum(t.astype(jnp.int32), NUM_OUTPUTS - 1)
            idxb[sl] = ix
            plsc.addupdate_scatter(hist, [ix], iones)

        @plsc.parallel_loop(0, ROW, step=L, unroll=16, carry=izeros)
        def _pc(off, mxc):
            sl = pl.ds(off, L)
            cg = plsc.load_gather(hist, [idxb[sl]])
            cntb[sl] = cg
            return jnp.maximum(mxc, cg)
        mf = jnp.full((L,), jnp.max(_pc), jnp.int32).astype(jnp.float32)

        @plsc.parallel_loop(0, ROW, step=L, unroll=16, carry=(zeros, zeros))
        def _pd(off, carry):
            sv, nv = carry
            sl = pl.ds(off, L)
            cg = cntb[sl]
            e = jnp.exp(cg.astype(jnp.float32) - mf)
            rc = plsc.load_gather(rtab, [cg])
            ebuf[sl] = e
            plsc.store_scatter(hist, [idxb[sl]], izeros)
            return sv + e * rc, nv + rc
        sv, nv = _pd
        s_occ = jnp.sum(sv)
        nnz = jnp.sum(nv)

        emv = jnp.exp(-mf)
        denom = (float(NUM_OUTPUTS) - nnz) * emv + s_occ
        s0 = float(MULT) * emv / denom
        tmul = float(MULT) / denom

        if out_pending is None:
            _drain_out()
        else:
            @pl.when(out_pending)
            def _():
                _drain_out()

        @plsc.parallel_loop(0, NUM_OUTPUTS, step=L, unroll=16)
        def _pe(off):
            sl = pl.ds(off, L)
            orow[sl] = xr[sl] * s0

        @plsc.parallel_loop(0, ROW, step=L, unroll=16)
        def _pf(off):
            sl = pl.ds(off, L)
            ix = idxb[sl]
            xg = plsc.load_gather(xr, [ix])
            plsc.store_scatter(orow, [ix], xg * ebuf[sl] * tmul)

        pltpu.async_copy(orow, out_hbm.at[r], sem_o)

    def _pair(g, _):
        r0 = base + 2 * g
        j0 = 2 * g

        @pl.when(lax.rem(j0, EG) == 0)
        def _():
            _drain_eps()

            @pl.when(j0 // EG < NG - 1)
            def _():
                _start_eps(j0 // EG + 1)

        _drain_x(0)
        _start_x(r0 + 1, 1)
        _row(r0, j0, 0, g > 0)
        _drain_x(1)

        @pl.when(g < ROWS_PER_W // 2 - 1)
        def _():
            _start_x(r0 + 2, 0)
        _row(r0 + 1, j0 + 1, 1, None)
        return 0

    lax.fori_loop(0, ROWS_PER_W // 2, _pair, 0)
    _drain_out()


@jax.jit
def kernel(z_mean, z_var, x, epsilon):
    mesh = plsc.VectorSubcoreMesh(core_axis_name="c", subcore_axis_name="s",
                                  num_cores=2, num_subcores=16)
    run = pl.kernel(
        _body,
        out_type=jax.ShapeDtypeStruct((BATCH, NUM_OUTPUTS), jnp.float32),
        mesh=mesh,
        scratch_types=[
            pltpu.VMEM((ROWS_PER_W * DIM,), jnp.float32),
            pltpu.VMEM((ROWS_PER_W * DIM,), jnp.float32),
            pltpu.VMEM((2 * EGW,), jnp.float32),
            pltpu.VMEM((NUM_OUTPUTS,), jnp.float32),
            pltpu.VMEM((NUM_OUTPUTS,), jnp.float32),
            pltpu.VMEM((ROW,), jnp.float32),
            pltpu.VMEM((ROW,), jnp.int32),
            pltpu.VMEM((ROW,), jnp.int32),
            pltpu.VMEM((ROW,), jnp.float32),
            pltpu.VMEM((NUM_OUTPUTS,), jnp.float32),
            pltpu.VMEM((NUM_OUTPUTS,), jnp.int32),
            pltpu.VMEM((RT,), jnp.float32),
            pltpu.SemaphoreType.DMA,
            pltpu.SemaphoreType.DMA,
            pltpu.SemaphoreType.DMA,
            pltpu.SemaphoreType.DMA,
        ],
        compiler_params=pltpu.CompilerParams(needs_layout_passes=False),
        name="prob_dropout_sc",
    )
    return run(z_mean.reshape(-1), z_var.reshape(-1), x.reshape(-1),
               epsilon.reshape(-1))


# speedup vs baseline: 1.5907x; 1.5907x over previous
import jax
import jax.numpy as jnp
from jax import lax
from jax.experimental import pallas as pl
from jax.experimental.pallas import tpu as pltpu
from jax.experimental.pallas import tpu_sc as plsc

BATCH = 1024
DIM = 128
NUM_OUTPUTS = 16384
MULT = NUM_OUTPUTS // BATCH
ROW = DIM * MULT
NW = 32
ROWS_PER_W = BATCH // NW
L = 16
EG = 4
NG = ROWS_PER_W // EG
EGW = EG * ROW
RT = ROW + L


def _body(zm_hbm, zv_hbm, x_hbm, eps_hbm, out_hbm,
          zm_a, sf_a, epsg, xa, xb, zrow, idxb, cntb, ebuf, orow, hist, rtab,
          sem_e, sem_xa, sem_xb, sem_o):
    wid = lax.axis_index("s") * 2 + lax.axis_index("c")
    base = wid * ROWS_PER_W

    zeros = jnp.zeros((L,), jnp.float32)
    izeros = jnp.zeros((L,), jnp.int32)
    iones = jnp.full((L,), 1, jnp.int32)
    xrefs = (xa, xb)
    sem_x = (sem_xa, sem_xb)

    pltpu.sync_copy(zm_hbm.at[pl.ds(base * DIM, ROWS_PER_W * DIM)], zm_a)
    pltpu.sync_copy(zv_hbm.at[pl.ds(base * DIM, ROWS_PER_W * DIM)], sf_a)

    @plsc.parallel_loop(0, ROWS_PER_W * DIM, step=L, unroll=8)
    def _sf(off):
        sl = pl.ds(off, L)
        sf_a[sl] = jnp.exp(0.5 * sf_a[sl])

    @plsc.parallel_loop(0, NUM_OUTPUTS, step=L, unroll=8)
    def _clear(off):
        hist[pl.ds(off, L)] = izeros

    lanes = lax.iota(jnp.int32, L).astype(jnp.float32)

    @plsc.parallel_loop(0, RT, step=L, unroll=8)
    def _rt(off):
        c = jnp.full((L,), off, jnp.int32).astype(jnp.float32) + lanes
        rtab[pl.ds(off, L)] = 1.0 / c

    def _start_eps(g):
        pltpu.async_copy(eps_hbm.at[pl.ds((base + g * EG) * ROW, EGW)],
                         epsg.at[pl.ds(lax.rem(g, 2) * EGW, EGW)], sem_e)

    def _drain_eps():
        pltpu.make_async_copy(eps_hbm.at[pl.ds(0, EGW)],
                              epsg.at[pl.ds(0, EGW)], sem_e).wait()

    def _start_x(r, q):
        pltpu.async_copy(x_hbm.at[r], xrefs[q], sem_x[q])

    def _drain_x(q):
        pltpu.make_async_copy(x_hbm.at[0], xrefs[q], sem_x[q]).wait()

    def _drain_out():
        pltpu.make_async_copy(orow, out_hbm.at[0], sem_o).wait()

    _start_eps(0)
    _start_x(base, 0)

    def _row(r, j, q, out_pending):
        xr = xrefs[q]
        zm_base = j * DIM
        eps_base = lax.rem(j, 2 * EG) * ROW

        @plsc.parallel_loop(
            0, ROW, step=L, unroll=16,
            carry=(jnp.full((L,), jnp.inf, jnp.float32),
                   jnp.full((L,), -jnp.inf, jnp.float32)))
        def _pa(off, carry):
            mn, mx = carry
            dsl = pl.ds(zm_base + lax.rem(off, DIM), L)
            z = zm_a[dsl] + sf_a[dsl] * epsg[pl.ds(eps_base + off, L)]
            zrow[pl.ds(off, L)] = z
            return jnp.minimum(mn, z), jnp.maximum(mx, z)
        minv, maxv = _pa
        mn = jnp.min(minv)
        rng = jnp.maximum(jnp.max(maxv) - mn, 1e-12)
        scale = float(NUM_OUTPUTS) / jnp.full((L,), rng, jnp.float32)

        @plsc.parallel_loop(0, ROW, step=L, unroll=16)
        def _pb(off):
            sl = pl.ds(off, L)
            t = (zrow[sl] - mn) * scale
            ix = jnp.minimum(t.astype(jnp.int32), NUM_OUTPUTS - 1)
            idxb[sl] = ix
            plsc.addupdate_scatter(hist, [ix], iones)

        @plsc.parallel_loop(0, ROW, step=L, unroll=16, carry=izeros)
        def _pc(off, mxc):
            sl = pl.ds(off, L)
            cg = plsc.load_gather(hist, [idxb[sl]])
            cntb[sl] = cg
            return jnp.maximum(mxc, cg)
        mf = jnp.full((L,), jnp.max(_pc), jnp.int32).astype(jnp.float32)

        @plsc.parallel_loop(0, ROW, step=L, unroll=16, carry=(zeros, zeros))
        def _pd(off, carry):
            sv, nv = carry
            sl = pl.ds(off, L)
            cg = cntb[sl]
            e = jnp.exp(cg.astype(jnp.float32) - mf)
            rc = plsc.load_gather(rtab, [cg])
            ebuf[sl] = e
            plsc.store_scatter(hist, [idxb[sl]], izeros)
            return sv + e * rc, nv + rc
        sv, nv = _pd
        s_occ = jnp.sum(sv)
        nnz = jnp.sum(nv)

        emv = jnp.exp(-mf)
        denom = (float(NUM_OUTPUTS) - nnz) * emv + s_occ
        s0 = float(MULT) * emv / denom
        tmul = float(MULT) / denom

        if out_pending is None:
            _drain_out()
        else:
            @pl.when(out_pending)
            def _():
                _drain_out()

        @plsc.parallel_loop(0, NUM_OUTPUTS, step=L, unroll=16)
        def _pe(off):
            sl = pl.ds(off, L)
            orow[sl] = xr[sl] * s0

        @plsc.parallel_loop(0, ROW, step=L, unroll=16)
        def _pf(off):
            sl = pl.ds(off, L)
            ix = idxb[sl]
            xg = plsc.load_gather(xr, [ix])
            plsc.store_scatter(orow, [ix], xg * ebuf[sl] * tmul)

        pltpu.async_copy(orow, out_hbm.at[r], sem_o)

    def _pair(g, _):
        r0 = base + 2 * g
        j0 = 2 * g

        @pl.when(lax.rem(j0, EG) == 0)
        def _():
            _drain_eps()

            @pl.when(j0 // EG < NG - 1)
            def _():
                _start_eps(j0 // EG + 1)

        _drain_x(0)
        _start_x(r0 + 1, 1)
        _row(r0, j0, 0, g > 0)
        _drain_x(1)

        @pl.when(g < ROWS_PER_W // 2 - 1)
        def _():
            _start_x(r0 + 2, 0)
        _row(r0 + 1, j0 + 1, 1, None)
        return 0

    lax.fori_loop(0, ROWS_PER_W // 2, _pair, 0)
    _drain_out()


@jax.jit
def kernel(z_mean, z_var, x, epsilon):
    mesh = plsc.VectorSubcoreMesh(core_axis_name="c", subcore_axis_name="s",
                                  num_cores=2, num_subcores=16)
    run = pl.kernel(
        _body,
        out_type=jax.ShapeDtypeStruct((BATCH, NUM_OUTPUTS), jnp.float32),
        mesh=mesh,
        scratch_types=[
            pltpu.VMEM((ROWS_PER_W * DIM,), jnp.float32),
            pltpu.VMEM((ROWS_PER_W * DIM,), jnp.float32),
            pltpu.VMEM((2 * EGW,), jnp.float32),
            pltpu.VMEM((NUM_OUTPUTS,), jnp.float32),
            pltpu.VMEM((NUM_OUTPUTS,), jnp.float32),
            pltpu.VMEM((ROW,), jnp.float32),
            pltpu.VMEM((ROW,), jnp.int32),
            pltpu.VMEM((ROW,), jnp.int32),
            pltpu.VMEM((ROW,), jnp.float32),
            pltpu.VMEM((NUM_OUTPUTS,), jnp.float32),
            pltpu.VMEM((NUM_OUTPUTS,), jnp.int32),
            pltpu.VMEM((RT,), jnp.float32),
            pltpu.SemaphoreType.DMA,
            pltpu.SemaphoreType.DMA,
            pltpu.SemaphoreType.DMA,
            pltpu.SemaphoreType.DMA,
        ],
        compiler_params=pltpu.CompilerParams(needs_layout_passes=False),
        name="prob_dropout_sc",
    )
    return run(z_mean.reshape(-1), z_var.reshape(-1), x, epsilon.reshape(-1))


# speedup vs baseline: 1.5917x; 1.0006x over previous
import jax
import jax.numpy as jnp
from jax import lax
from jax.experimental import pallas as pl
from jax.experimental.pallas import tpu as pltpu
from jax.experimental.pallas import tpu_sc as plsc

BATCH = 1024
DIM = 128
NUM_OUTPUTS = 16384
MULT = NUM_OUTPUTS // BATCH
ROW = DIM * MULT
NW = 32
ROWS_PER_W = BATCH // NW
L = 16
EG = 4
NG = ROWS_PER_W // EG
EGW = EG * ROW
RT = ROW + L


def _body(zm_hbm, zv_hbm, x_hbm, eps_hbm, out_hbm,
          zm_a, sf_a, epsg, xa, xb, zrow, idxb, cntb, ebuf, oa, ob, hist,
          rtab, sem_e, sem_xa, sem_xb, sem_oa, sem_ob):
    wid = lax.axis_index("s") * 2 + lax.axis_index("c")
    base = wid * ROWS_PER_W

    zeros = jnp.zeros((L,), jnp.float32)
    izeros = jnp.zeros((L,), jnp.int32)
    iones = jnp.full((L,), 1, jnp.int32)
    xrefs = (xa, xb)
    sem_x = (sem_xa, sem_xb)
    orefs = (oa, ob)
    sem_o = (sem_oa, sem_ob)

    pltpu.sync_copy(zm_hbm.at[pl.ds(base * DIM, ROWS_PER_W * DIM)], zm_a)
    pltpu.sync_copy(zv_hbm.at[pl.ds(base * DIM, ROWS_PER_W * DIM)], sf_a)

    @plsc.parallel_loop(0, ROWS_PER_W * DIM, step=L, unroll=8)
    def _sf(off):
        sl = pl.ds(off, L)
        sf_a[sl] = jnp.exp(0.5 * sf_a[sl])

    @plsc.parallel_loop(0, NUM_OUTPUTS, step=L, unroll=8)
    def _clear(off):
        hist[pl.ds(off, L)] = izeros

    lanes = lax.iota(jnp.int32, L).astype(jnp.float32)

    @plsc.parallel_loop(0, RT, step=L, unroll=8)
    def _rt(off):
        c = jnp.full((L,), off, jnp.int32).astype(jnp.float32) + lanes
        rtab[pl.ds(off, L)] = 1.0 / c

    def _start_eps(g):
        pltpu.async_copy(eps_hbm.at[pl.ds((base + g * EG) * ROW, EGW)],
                         epsg.at[pl.ds(lax.rem(g, 2) * EGW, EGW)], sem_e)

    def _drain_eps():
        pltpu.make_async_copy(eps_hbm.at[pl.ds(0, EGW)],
                              epsg.at[pl.ds(0, EGW)], sem_e).wait()

    def _start_x(r, q):
        pltpu.async_copy(x_hbm.at[r], xrefs[q], sem_x[q])

    def _drain_x(q):
        pltpu.make_async_copy(x_hbm.at[0], xrefs[q], sem_x[q]).wait()

    def _drain_out(q):
        pltpu.make_async_copy(orefs[q], out_hbm.at[0], sem_o[q]).wait()

    _start_eps(0)
    _start_x(base, 0)

    def _row(r, j, q, out_pending):
        xr = xrefs[q]
        orow = orefs[q]
        zm_base = j * DIM
        eps_base = lax.rem(j, 2 * EG) * ROW

        @plsc.parallel_loop(
            0, ROW, step=L, unroll=16,
            carry=(jnp.full((L,), jnp.inf, jnp.float32),
                   jnp.full((L,), -jnp.inf, jnp.float32)))
        def _pa(off, carry):
            mn, mx = carry
            dsl = pl.ds(zm_base + lax.rem(off, DIM), L)
            z = zm_a[dsl] + sf_a[dsl] * epsg[pl.ds(eps_base + off, L)]
            zrow[pl.ds(off, L)] = z
            return jnp.minimum(mn, z), jnp.maximum(mx, z)
        minv, maxv = _pa
        mn = jnp.min(minv)
        rng = jnp.maximum(jnp.max(maxv) - mn, 1e-12)
        scale = float(NUM_OUTPUTS) / jnp.full((L,), rng, jnp.float32)

        @plsc.parallel_loop(0, ROW, step=L, unroll=16)
        def _pb(off):
            sl = pl.ds(off, L)
            t = (zrow[sl] - mn) * scale
            ix = jnp.minimum(t.astype(jnp.int32), NUM_OUTPUTS - 1)
            idxb[sl] = ix
            plsc.addupdate_scatter(hist, [ix], iones)

        @plsc.parallel_loop(0, ROW, step=L, unroll=16, carry=izeros)
        def _pc(off, mxc):
            sl = pl.ds(off, L)
            cg = plsc.load_gather(hist, [idxb[sl]])
            cntb[sl] = cg
            return jnp.maximum(mxc, cg)
        mf = jnp.full((L,), jnp.max(_pc), jnp.int32).astype(jnp.float32)

        @plsc.parallel_loop(0, ROW, step=L, unroll=16, carry=(zeros, zeros))
        def _pd(off, carry):
            sv, nv = carry
            sl = pl.ds(off, L)
            cg = cntb[sl]
            e = jnp.exp(cg.astype(jnp.float32) - mf)
            rc = plsc.load_gather(rtab, [cg])
            ebuf[sl] = e
            plsc.store_scatter(hist, [idxb[sl]], izeros)
            return sv + e * rc, nv + rc
        sv, nv = _pd
        s_occ = jnp.sum(sv)
        nnz = jnp.sum(nv)

        emv = jnp.exp(-mf)
        denom = (float(NUM_OUTPUTS) - nnz) * emv + s_occ
        s0 = float(MULT) * emv / denom
        tmul = float(MULT) / denom

        @pl.when(out_pending)
        def _():
            _drain_out(q)

        @plsc.parallel_loop(0, NUM_OUTPUTS, step=L, unroll=16)
        def _pe(off):
            sl = pl.ds(off, L)
            orow[sl] = xr[sl] * s0

        @plsc.parallel_loop(0, ROW, step=L, unroll=16)
        def _pf(off):
            sl = pl.ds(off, L)
            ix = idxb[sl]
            xg = plsc.load_gather(xr, [ix])
            plsc.store_scatter(orow, [ix], xg * ebuf[sl] * tmul)

        pltpu.async_copy(orow, out_hbm.at[r], sem_o[q])

    def _pair(g, _):
        r0 = base + 2 * g
        j0 = 2 * g

        @pl.when(lax.rem(j0, EG) == 0)
        def _():
            _drain_eps()

            @pl.when(j0 // EG < NG - 1)
            def _():
                _start_eps(j0 // EG + 1)

        _drain_x(0)
        _start_x(r0 + 1, 1)
        _row(r0, j0, 0, g > 0)
        _drain_x(1)

        @pl.when(g < ROWS_PER_W // 2 - 1)
        def _():
            _start_x(r0 + 2, 0)
        _row(r0 + 1, j0 + 1, 1, g > 0)
        return 0

    lax.fori_loop(0, ROWS_PER_W // 2, _pair, 0)
    _drain_out(0)
    _drain_out(1)


@jax.jit
def kernel(z_mean, z_var, x, epsilon):
    mesh = plsc.VectorSubcoreMesh(core_axis_name="c", subcore_axis_name="s",
                                  num_cores=2, num_subcores=16)
    run = pl.kernel(
        _body,
        out_type=jax.ShapeDtypeStruct((BATCH, NUM_OUTPUTS), jnp.float32),
        mesh=mesh,
        scratch_types=[
            pltpu.VMEM((ROWS_PER_W * DIM,), jnp.float32),
            pltpu.VMEM((ROWS_PER_W * DIM,), jnp.float32),
            pltpu.VMEM((2 * EGW,), jnp.float32),
            pltpu.VMEM((NUM_OUTPUTS,), jnp.float32),
            pltpu.VMEM((NUM_OUTPUTS,), jnp.float32),
            pltpu.VMEM((ROW,), jnp.float32),
            pltpu.VMEM((ROW,), jnp.int32),
            pltpu.VMEM((ROW,), jnp.int32),
            pltpu.VMEM((ROW,), jnp.float32),
            pltpu.VMEM((NUM_OUTPUTS,), jnp.float32),
            pltpu.VMEM((NUM_OUTPUTS,), jnp.float32),
            pltpu.VMEM((NUM_OUTPUTS,), jnp.int32),
            pltpu.VMEM((RT,), jnp.float32),
            pltpu.SemaphoreType.DMA,
            pltpu.SemaphoreType.DMA,
            pltpu.SemaphoreType.DMA,
            pltpu.SemaphoreType.DMA,
            pltpu.SemaphoreType.DMA,
        ],
        compiler_params=pltpu.CompilerParams(needs_layout_passes=False),
        name="prob_dropout_sc",
    )
    return run(z_mean.reshape(-1), z_var.reshape(-1), x, epsilon.reshape(-1))


# speedup vs baseline: 1.5947x; 1.0019x over previous
import jax
import jax.numpy as jnp
from jax import lax
from jax.experimental import pallas as pl
from jax.experimental.pallas import tpu as pltpu
from jax.experimental.pallas import tpu_sc as plsc

BATCH = 1024
DIM = 128
NUM_OUTPUTS = 16384
MULT = NUM_OUTPUTS // BATCH
ROW = DIM * MULT
NW = 32
ROWS_PER_W = BATCH // NW
L = 16
EG = 4
NG = ROWS_PER_W // EG
EGW = EG * ROW
RT = ROW + L


def _body(zm_hbm, zv_hbm, x_hbm, eps_hbm, out_hbm,
          zm_a, sf_a, epsg, xa, xb, zrow, idxb, cntb, ebuf, oa, ob, hist,
          rtab, sem_e, sem_xa, sem_xb, sem_oa, sem_ob):
    wid = lax.axis_index("s") * 2 + lax.axis_index("c")
    base = wid * ROWS_PER_W

    zeros = jnp.zeros((L,), jnp.float32)
    izeros = jnp.zeros((L,), jnp.int32)
    iones = jnp.full((L,), 1, jnp.int32)
    xrefs = (xa, xb)
    sem_x = (sem_xa, sem_xb)
    orefs = (oa, ob)
    sem_o = (sem_oa, sem_ob)

    pltpu.sync_copy(zm_hbm.at[pl.ds(base * DIM, ROWS_PER_W * DIM)], zm_a)
    pltpu.sync_copy(zv_hbm.at[pl.ds(base * DIM, ROWS_PER_W * DIM)], sf_a)

    @plsc.parallel_loop(0, ROWS_PER_W * DIM, step=L, unroll=8)
    def _sf(off):
        sl = pl.ds(off, L)
        sf_a[sl] = jnp.exp(0.5 * sf_a[sl])

    @plsc.parallel_loop(0, NUM_OUTPUTS, step=L, unroll=8)
    def _clear(off):
        hist[pl.ds(off, L)] = izeros

    lanes = lax.iota(jnp.int32, L).astype(jnp.float32)

    @plsc.parallel_loop(0, RT, step=L, unroll=8)
    def _rt(off):
        c = jnp.full((L,), off, jnp.int32).astype(jnp.float32) + lanes
        rtab[pl.ds(off, L)] = 1.0 / c

    def _start_eps(g):
        pltpu.async_copy(eps_hbm.at[pl.ds((base + g * EG) * ROW, EGW)],
                         epsg.at[pl.ds(lax.rem(g, 2) * EGW, EGW)], sem_e)

    def _drain_eps():
        pltpu.make_async_copy(eps_hbm.at[pl.ds(0, EGW)],
                              epsg.at[pl.ds(0, EGW)], sem_e).wait()

    def _start_x(r, q):
        pltpu.async_copy(x_hbm.at[r], xrefs[q], sem_x[q])

    def _drain_x(q):
        pltpu.make_async_copy(x_hbm.at[0], xrefs[q], sem_x[q]).wait()

    def _drain_out(q):
        pltpu.make_async_copy(orefs[q], out_hbm.at[0], sem_o[q]).wait()

    _start_eps(0)
    _start_x(base, 0)

    def _row(r, j, q, out_pending):
        xr = xrefs[q]
        orow = orefs[q]
        zm_base = j * DIM
        eps_base = lax.rem(j, 2 * EG) * ROW

        @plsc.parallel_loop(
            0, ROW, step=L, unroll=16,
            carry=(jnp.full((L,), jnp.inf, jnp.float32),
                   jnp.full((L,), -jnp.inf, jnp.float32)))
        def _pa(off, carry):
            mn, mx = carry
            dsl = pl.ds(zm_base + lax.rem(off, DIM), L)
            z = zm_a[dsl] + sf_a[dsl] * epsg[pl.ds(eps_base + off, L)]
            zrow[pl.ds(off, L)] = z
            return jnp.minimum(mn, z), jnp.maximum(mx, z)
        minv, maxv = _pa
        mn = jnp.min(minv)
        rng = jnp.maximum(jnp.max(maxv) - mn, 1e-12)
        scale = float(NUM_OUTPUTS) / jnp.full((L,), rng, jnp.float32)

        @plsc.parallel_loop(0, ROW, step=L, unroll=16)
        def _pb(off):
            sl = pl.ds(off, L)
            t = (zrow[sl] - mn) * scale
            ix = jnp.minimum(t.astype(jnp.int32), NUM_OUTPUTS - 1)
            idxb[sl] = ix
            plsc.addupdate_scatter(hist, [ix], iones)

        @plsc.parallel_loop(0, ROW, step=L, unroll=16, carry=izeros)
        def _pc(off, mxc):
            sl = pl.ds(off, L)
            cg = plsc.load_gather(hist, [idxb[sl]])
            cntb[sl] = cg
            return jnp.maximum(mxc, cg)
        mf = jnp.full((L,), jnp.max(_pc), jnp.int32).astype(jnp.float32)
        emv = jnp.exp(-mf)

        @plsc.parallel_loop(0, ROW, step=L, unroll=16, carry=zeros)
        def _pd(off, sv):
            sl = pl.ds(off, L)
            cg = cntb[sl]
            e = jnp.exp(cg.astype(jnp.float32) - mf)
            rc = plsc.load_gather(rtab, [cg])
            ebuf[sl] = e
            plsc.store_scatter(hist, [idxb[sl]], izeros)
            return sv + (e - emv) * rc
        denom = float(NUM_OUTPUTS) * emv + jnp.sum(_pd)
        s0 = float(MULT) * emv / denom
        tmul = float(MULT) / denom

        @pl.when(out_pending)
        def _():
            _drain_out(q)

        @plsc.parallel_loop(0, NUM_OUTPUTS, step=L, unroll=16)
        def _pe(off):
            sl = pl.ds(off, L)
            orow[sl] = xr[sl] * s0

        @plsc.parallel_loop(0, ROW, step=L, unroll=16)
        def _pf(off):
            sl = pl.ds(off, L)
            ix = idxb[sl]
            xg = plsc.load_gather(xr, [ix])
            plsc.store_scatter(orow, [ix], xg * ebuf[sl] * tmul)

        pltpu.async_copy(orow, out_hbm.at[r], sem_o[q])

    def _pair(g, _):
        r0 = base + 2 * g
        j0 = 2 * g

        @pl.when(lax.rem(j0, EG) == 0)
        def _():
            _drain_eps()

            @pl.when(j0 // EG < NG - 1)
            def _():
                _start_eps(j0 // EG + 1)

        _drain_x(0)
        _start_x(r0 + 1, 1)
        _row(r0, j0, 0, g > 0)
        _drain_x(1)

        @pl.when(g < ROWS_PER_W // 2 - 1)
        def _():
            _start_x(r0 + 2, 0)
        _row(r0 + 1, j0 + 1, 1, g > 0)
        return 0

    lax.fori_loop(0, ROWS_PER_W // 2, _pair, 0)
    _drain_out(0)
    _drain_out(1)


@jax.jit
def kernel(z_mean, z_var, x, epsilon):
    mesh = plsc.VectorSubcoreMesh(core_axis_name="c", subcore_axis_name="s",
                                  num_cores=2, num_subcores=16)
    run = pl.kernel(
        _body,
        out_type=jax.ShapeDtypeStruct((BATCH, NUM_OUTPUTS), jnp.float32),
        mesh=mesh,
        scratch_types=[
            pltpu.VMEM((ROWS_PER_W * DIM,), jnp.float32),
            pltpu.VMEM((ROWS_PER_W * DIM,), jnp.float32),
            pltpu.VMEM((2 * EGW,), jnp.float32),
            pltpu.VMEM((NUM_OUTPUTS,), jnp.float32),
            pltpu.VMEM((NUM_OUTPUTS,), jnp.float32),
            pltpu.VMEM((ROW,), jnp.float32),
            pltpu.VMEM((ROW,), jnp.int32),
            pltpu.VMEM((ROW,), jnp.int32),
            pltpu.VMEM((ROW,), jnp.float32),
            pltpu.VMEM((NUM_OUTPUTS,), jnp.float32),
            pltpu.VMEM((NUM_OUTPUTS,), jnp.float32),
            pltpu.VMEM((NUM_OUTPUTS,), jnp.int32),
            pltpu.VMEM((RT,), jnp.float32),
            pltpu.SemaphoreType.DMA,
            pltpu.SemaphoreType.DMA,
            pltpu.SemaphoreType.DMA,
            pltpu.SemaphoreType.DMA,
            pltpu.SemaphoreType.DMA,
        ],
        compiler_params=pltpu.CompilerParams(needs_layout_passes=False),
        name="prob_dropout_sc",
    )
    return run(z_mean.reshape(-1), z_var.reshape(-1), x, epsilon.reshape(-1))


# speedup vs baseline: 1.6185x; 1.0149x over previous
import jax
import jax.numpy as jnp
from jax import lax
from jax.experimental import pallas as pl
from jax.experimental.pallas import tpu as pltpu
from jax.experimental.pallas import tpu_sc as plsc

BATCH = 1024
DIM = 128
NUM_OUTPUTS = 16384
MULT = NUM_OUTPUTS // BATCH
ROW = DIM * MULT
NW = 32
ROWS_PER_W = BATCH // NW
L = 16
EG = 4
NG = ROWS_PER_W // EG
EGW = EG * ROW
RT = ROW + L


def _body(zm_hbm, zv_hbm, x_hbm, eps_hbm, out_hbm,
          zm_a, sf_a, epsg, xa, xb, zrow, idxb, cntb, ebuf, oa, ob, hist,
          rtab, sem_e, sem_xa, sem_xb, sem_oa, sem_ob):
    wid = lax.axis_index("s") * 2 + lax.axis_index("c")
    base = wid * ROWS_PER_W

    zeros = jnp.zeros((L,), jnp.float32)
    izeros = jnp.zeros((L,), jnp.int32)
    iones = jnp.full((L,), 1, jnp.int32)
    xrefs = (xa, xb)
    sem_x = (sem_xa, sem_xb)
    orefs = (oa, ob)
    sem_o = (sem_oa, sem_ob)

    pltpu.async_copy(zm_hbm.at[pl.ds(base * DIM, ROWS_PER_W * DIM)], zm_a,
                     sem_oa)
    pltpu.async_copy(zv_hbm.at[pl.ds(base * DIM, ROWS_PER_W * DIM)], sf_a,
                     sem_oa)

    @plsc.parallel_loop(0, NUM_OUTPUTS, step=L, unroll=8)
    def _clear(off):
        hist[pl.ds(off, L)] = izeros

    lanes = lax.iota(jnp.int32, L).astype(jnp.float32)

    @plsc.parallel_loop(0, RT, step=L, unroll=8)
    def _rt(off):
        c = jnp.full((L,), off, jnp.int32).astype(jnp.float32) + lanes
        rtab[pl.ds(off, L)] = 1.0 / c

    pltpu.make_async_copy(zm_hbm.at[pl.ds(0, ROWS_PER_W * DIM)], zm_a,
                          sem_oa).wait()
    pltpu.make_async_copy(zv_hbm.at[pl.ds(0, ROWS_PER_W * DIM)], sf_a,
                          sem_oa).wait()

    @plsc.parallel_loop(0, ROWS_PER_W * DIM, step=L, unroll=8)
    def _sf(off):
        sl = pl.ds(off, L)
        sf_a[sl] = jnp.exp(0.5 * sf_a[sl])

    def _start_eps(g):
        pltpu.async_copy(eps_hbm.at[pl.ds((base + g * EG) * ROW, EGW)],
                         epsg.at[pl.ds(lax.rem(g, 2) * EGW, EGW)], sem_e)

    def _drain_eps():
        pltpu.make_async_copy(eps_hbm.at[pl.ds(0, EGW)],
                              epsg.at[pl.ds(0, EGW)], sem_e).wait()

    def _start_x(r, q):
        pltpu.async_copy(x_hbm.at[r], xrefs[q], sem_x[q])

    def _drain_x(q):
        pltpu.make_async_copy(x_hbm.at[0], xrefs[q], sem_x[q]).wait()

    def _drain_out(q):
        pltpu.make_async_copy(orefs[q], out_hbm.at[0], sem_o[q]).wait()

    _start_eps(0)
    _start_x(base, 0)

    def _row(r, j, q, out_pending):
        xr = xrefs[q]
        orow = orefs[q]
        zm_base = j * DIM
        eps_base = lax.rem(j, 2 * EG) * ROW

        @plsc.parallel_loop(
            0, ROW, step=L, unroll=16,
            carry=(jnp.full((L,), jnp.inf, jnp.float32),
                   jnp.full((L,), -jnp.inf, jnp.float32)))
        def _pa(off, carry):
            mn, mx = carry
            dsl = pl.ds(zm_base + lax.rem(off, DIM), L)
            z = zm_a[dsl] + sf_a[dsl] * epsg[pl.ds(eps_base + off, L)]
            zrow[pl.ds(off, L)] = z
            return jnp.minimum(mn, z), jnp.maximum(mx, z)
        minv, maxv = _pa
        mn = jnp.min(minv)
        rng = jnp.maximum(jnp.max(maxv) - mn, 1e-12)
        scale = float(NUM_OUTPUTS) / jnp.full((L,), rng, jnp.float32)

        @plsc.parallel_loop(0, ROW, step=L, unroll=16)
        def _pb(off):
            sl = pl.ds(off, L)
            t = (zrow[sl] - mn) * scale
            ix = jnp.minimum(t.astype(jnp.int32), NUM_OUTPUTS - 1)
            idxb[sl] = ix
            plsc.addupdate_scatter(hist, [ix], iones)

        @plsc.parallel_loop(0, ROW, step=L, unroll=16, carry=izeros)
        def _pc(off, mxc):
            sl = pl.ds(off, L)
            cg = plsc.load_gather(hist, [idxb[sl]])
            cntb[sl] = cg
            return jnp.maximum(mxc, cg)
        mf = jnp.full((L,), jnp.max(_pc), jnp.int32).astype(jnp.float32)
        emv = jnp.exp(-mf)

        @plsc.parallel_loop(0, ROW, step=L, unroll=16, carry=zeros)
        def _pd(off, sv):
            sl = pl.ds(off, L)
            cg = cntb[sl]
            e = jnp.exp(cg.astype(jnp.float32) - mf)
            rc = plsc.load_gather(rtab, [cg])
            ebuf[sl] = e
            plsc.store_scatter(hist, [idxb[sl]], izeros)
            return sv + (e - emv) * rc
        denom = float(NUM_OUTPUTS) * emv + jnp.sum(_pd)
        s0 = float(MULT) * emv / denom
        tmul = float(MULT) / denom

        @pl.when(out_pending)
        def _():
            _drain_out(q)

        @plsc.parallel_loop(0, NUM_OUTPUTS, step=L, unroll=16)
        def _pe(off):
            sl = pl.ds(off, L)
            orow[sl] = xr[sl] * s0

        @plsc.parallel_loop(0, ROW, step=L, unroll=16)
        def _pf(off):
            sl = pl.ds(off, L)
            ix = idxb[sl]
            xg = plsc.load_gather(xr, [ix])
            plsc.store_scatter(orow, [ix], xg * ebuf[sl] * tmul)

        pltpu.async_copy(orow, out_hbm.at[r], sem_o[q])

    def _pair(g, _):
        r0 = base + 2 * g
        j0 = 2 * g

        @pl.when(lax.rem(j0, EG) == 0)
        def _():
            _drain_eps()

            @pl.when(j0 // EG < NG - 1)
            def _():
                _start_eps(j0 // EG + 1)

        _drain_x(0)
        _start_x(r0 + 1, 1)
        _row(r0, j0, 0, g > 0)
        _drain_x(1)

        @pl.when(g < ROWS_PER_W // 2 - 1)
        def _():
            _start_x(r0 + 2, 0)
        _row(r0 + 1, j0 + 1, 1, g > 0)
        return 0

    lax.fori_loop(0, ROWS_PER_W // 2, _pair, 0)
    _drain_out(0)
    _drain_out(1)


@jax.jit
def kernel(z_mean, z_var, x, epsilon):
    mesh = plsc.VectorSubcoreMesh(core_axis_name="c", subcore_axis_name="s",
                                  num_cores=2, num_subcores=16)
    run = pl.kernel(
        _body,
        out_type=jax.ShapeDtypeStruct((BATCH, NUM_OUTPUTS), jnp.float32),
        mesh=mesh,
        scratch_types=[
            pltpu.VMEM((ROWS_PER_W * DIM,), jnp.float32),
            pltpu.VMEM((ROWS_PER_W * DIM,), jnp.float32),
            pltpu.VMEM((2 * EGW,), jnp.float32),
            pltpu.VMEM((NUM_OUTPUTS,), jnp.float32),
            pltpu.VMEM((NUM_OUTPUTS,), jnp.float32),
            pltpu.VMEM((ROW,), jnp.float32),
            pltpu.VMEM((ROW,), jnp.int32),
            pltpu.VMEM((ROW,), jnp.int32),
            pltpu.VMEM((ROW,), jnp.float32),
            pltpu.VMEM((NUM_OUTPUTS,), jnp.float32),
            pltpu.VMEM((NUM_OUTPUTS,), jnp.float32),
            pltpu.VMEM((NUM_OUTPUTS,), jnp.int32),
            pltpu.VMEM((RT,), jnp.float32),
            pltpu.SemaphoreType.DMA,
            pltpu.SemaphoreType.DMA,
            pltpu.SemaphoreType.DMA,
            pltpu.SemaphoreType.DMA,
            pltpu.SemaphoreType.DMA,
        ],
        compiler_params=pltpu.CompilerParams(needs_layout_passes=False),
        name="prob_dropout_sc",
    )
    return run(z_mean.reshape(-1), z_var.reshape(-1), x, epsilon.reshape(-1))
